# baseline (speedup 1.0000x reference)
import functools

import jax
import jax.numpy as jnp
from jax import lax
from jax.experimental import pallas as pl
from jax.experimental.pallas import tpu as pltpu
from jax.experimental.pallas import tpu_sc as plsc

_INFO = plsc.get_sparse_core_info()
_NC = _INFO.num_cores
_NS = _INFO.num_subcores
_NW = _NC * _NS
_L = _INFO.num_lanes


def _gather_call(hh, bb, vv, dd):
    sub = dd // 8
    bt = bb // 128
    mesh = plsc.VectorSubcoreMesh(core_axis_name="c", subcore_axis_name="s")

    @functools.partial(
        pl.kernel,
        mesh=mesh,
        compiler_params=pltpu.CompilerParams(
            use_tc_tiling_on_sc=False, needs_layout_passes=False),
        out_type=jax.ShapeDtypeStruct((hh, sub, bt, 8 * 128), jnp.float32),
        scratch_types=[
            pltpu.VMEM((vv,), jnp.float32),
            pltpu.VMEM((2, bt, 128), jnp.int32),
            pltpu.VMEM((4, bt, 128), jnp.float32),
            pltpu.SemaphoreType.DMA,
            pltpu.SemaphoreType.DMA,
            pltpu.SemaphoreType.DMA,
        ],
    )
    def grab(ids_hbm, tab_hbm, out_hbm, tab_v, idx_v, out_v, sem_t, sem_i, sem_o):
        w = lax.axis_index("s") * _NC + lax.axis_index("c")
        tr = w // 8
        r = w % 8

        def idx_row(h):
            return ids_hbm.at[h // 8, :, h % 8, :]

        pltpu.async_copy(tab_hbm.at[w], tab_v, sem_t)
        pltpu.async_copy(idx_row(0), idx_v.at[0], sem_i)
        pltpu.async_copy(idx_row(1), idx_v.at[1], sem_i)
        pltpu.make_async_copy(tab_hbm.at[w], tab_v, sem_t).wait()

        def idx_wait():
            pltpu.make_async_copy(idx_row(0), idx_v.at[0], sem_i).wait()

        def store_wait():
            pltpu.make_async_copy(
                out_v.at[0], out_hbm.at[0, 0, :, pl.ds(0, 128)], sem_o).wait()

        def do_h(h, islot, oslot, first):
            idx_wait()
            if not first:
                store_wait()

            @plsc.parallel_loop(0, bt, unroll=8)
            def rowb(tc):
                for k in range(128 // _L):
                    iv = idx_v[islot, tc, pl.ds(k * _L, _L)]
                    vals = plsc.load_gather(tab_v, [iv])
                    out_v[oslot, tc, pl.ds(k * _L, _L)] = vals

            @pl.when(h + 2 < hh)
            def _():
                pltpu.async_copy(idx_row(h + 2), idx_v.at[islot], sem_i)

            pltpu.async_copy(
                out_v.at[oslot], out_hbm.at[h, tr, :, pl.ds(r * 128, 128)],
                sem_o)

        def hquad(q, _):
            for j in range(4):
                do_h(4 * q + j, j % 2, j, first=False)
            return 0

        for j in range(4):
            do_h(j, j % 2, j, first=True)
        lax.fori_loop(1, hh // 4, hquad, 0)
        for _ in range(4):
            store_wait()

    return grab


def kernel(prompt_token_ids, table):
    b, h = prompt_token_ids.shape
    v, d = table.shape
    ids_4d = (prompt_token_ids.astype(jnp.int32).T
              .reshape(h // 8, 8, b // 128, 128).transpose(0, 2, 1, 3))
    table_t = table.T
    out = _gather_call(h, b, v, d)(ids_4d, table_t)
    out = out.reshape(h, d // 8, b // 128, 8, 128)
    return out.transpose(2, 4, 0, 1, 3).reshape(b, h, d)


# speedup vs baseline: 1.6997x; 1.6997x over previous
import functools

import jax
import jax.numpy as jnp
from jax import lax
from jax.experimental import pallas as pl
from jax.experimental.pallas import tpu as pltpu
from jax.experimental.pallas import tpu_sc as plsc

_INFO = plsc.get_sparse_core_info()
_NC = _INFO.num_cores
_NS = _INFO.num_subcores
_NW = _NC * _NS
_L = _INFO.num_lanes
_CH = 20


def _gather_call(hh, bb, vv, dd):
    sub = dd // 8
    bt = bb // 128
    n_chunks = hh // _CH
    mesh = plsc.VectorSubcoreMesh(core_axis_name="c", subcore_axis_name="s")

    @functools.partial(
        pl.kernel,
        mesh=mesh,
        compiler_params=pltpu.CompilerParams(
            use_tc_tiling_on_sc=False, needs_layout_passes=False),
        out_type=jax.ShapeDtypeStruct((hh, sub, bt, 8 * 128), jnp.float32),
        scratch_types=[
            pltpu.VMEM((vv,), jnp.float32),
            pltpu.VMEM((2, bt, 128), jnp.int32),
            pltpu.VMEM((2, bt, 128), jnp.float32),
            pltpu.VMEM_SHARED((2, _CH, bt, 128), jnp.int32),
            pltpu.SemaphoreType.DMA,
            pltpu.SemaphoreType.DMA,
            pltpu.SemaphoreType.DMA,
            pltpu.SemaphoreType.DMA,
        ],
    )
    def grab(ids_hbm, tab_hbm, out_hbm, tab_v, idx_v, out_v, sids,
             sem_t, sem_i, sem_o, sem_s):
        c_ax = lax.axis_index("c")
        s_ax = lax.axis_index("s")
        w = s_ax * _NC + c_ax
        tr = w // 8
        r = w % 8
        lo = (_CH * s_ax) // _NS
        hi = (_CH * (s_ax + 1)) // _NS

        def idx_row(h):
            return ids_hbm.at[h // 8, :, h % 8, :]

        def stage(chunk, buf):
            def stg(i, _):
                pltpu.async_copy(idx_row(chunk * _CH + i), sids.at[buf, i],
                                 sem_s)
                return 0

            lax.fori_loop(lo, hi, stg, 0)

        def stage_sync():
            def stw(i, _):
                pltpu.make_async_copy(idx_row(0), sids.at[0, 0], sem_s).wait()
                return 0

            lax.fori_loop(lo, hi, stw, 0)
            plsc.subcore_barrier()

        def idx_fetch(buf, i, islot):
            pltpu.async_copy(sids.at[buf, i], idx_v.at[islot], sem_i)

        def idx_wait():
            pltpu.make_async_copy(sids.at[0, 0], idx_v.at[0], sem_i).wait()

        def store_wait():
            pltpu.make_async_copy(
                out_v.at[0], out_hbm.at[0, 0, :, pl.ds(0, 128)], sem_o).wait()

        pltpu.async_copy(tab_hbm.at[w], tab_v, sem_t)
        stage(0, 0)
        pltpu.make_async_copy(tab_hbm.at[w], tab_v, sem_t).wait()

        def do_h(h, buf, i, islot, first):
            idx_wait()
            if not first:
                store_wait()

            @plsc.parallel_loop(0, bt, unroll=8)
            def rowb(tc):
                for k in range(128 // _L):
                    iv = idx_v[islot, tc, pl.ds(k * _L, _L)]
                    vals = plsc.load_gather(tab_v, [iv])
                    out_v[islot, tc, pl.ds(k * _L, _L)] = vals

            @pl.when(i + 2 < _CH)
            def _():
                idx_fetch(buf, i + 2, islot)

            pltpu.async_copy(
                out_v.at[islot], out_hbm.at[h, tr, :, pl.ds(r * 128, 128)],
                sem_o)

        def do_chunk(chunk, buf, first):
            stage_sync()

            @pl.when(chunk + 1 < n_chunks)
            def _():
                stage(chunk + 1, 1 - buf)

            idx_fetch(buf, 0, 0)
            idx_fetch(buf, 1, 1)

            def pair(p, _):
                i0 = 2 * p
                h0 = chunk * _CH + i0
                do_h(h0, buf, i0, 0, first=False)
                do_h(h0 + 1, buf, i0 + 1, 1, first=False)
                return 0

            if first:
                do_h(chunk * _CH, buf, 0, 0, first=True)
                do_h(chunk * _CH + 1, buf, 1, 1, first=True)
                lax.fori_loop(1, _CH // 2, pair, 0)
            else:
                lax.fori_loop(0, _CH // 2, pair, 0)

        do_chunk(0, 0, first=True)

        def super_body(sc, _):
            do_chunk(2 * sc, 0, first=False)
            do_chunk(2 * sc + 1, 1, first=False)
            return 0

        do_chunk(1, 1, first=False)
        lax.fori_loop(1, n_chunks // 2, super_body, 0)
        store_wait()
        store_wait()

    return grab


def kernel(prompt_token_ids, table):
    b, h = prompt_token_ids.shape
    v, d = table.shape
    ids_4d = (prompt_token_ids.astype(jnp.int32).T
              .reshape(h // 8, 8, b // 128, 128).transpose(0, 2, 1, 3))
    table_t = table.T
    out = _gather_call(h, b, v, d)(ids_4d, table_t)
    out = out.reshape(h, d // 8, b // 128, 8, 128)
    return out.transpose(2, 4, 0, 1, 3).reshape(b, h, d)


# speedup vs baseline: 1.7158x; 1.0094x over previous
import functools

import jax
import jax.numpy as jnp
from jax import lax
from jax.experimental import pallas as pl
from jax.experimental.pallas import tpu as pltpu
from jax.experimental.pallas import tpu_sc as plsc

_INFO = plsc.get_sparse_core_info()
_NC = _INFO.num_cores
_NS = _INFO.num_subcores
_NW = _NC * _NS
_L = _INFO.num_lanes
_CH = 20


def _gather_call(hh, bb, vv, dd):
    sub = dd // 8
    bt = bb // 128
    n_chunks = hh // _CH
    mesh = plsc.VectorSubcoreMesh(core_axis_name="c", subcore_axis_name="s")

    @functools.partial(
        pl.kernel,
        mesh=mesh,
        compiler_params=pltpu.CompilerParams(
            use_tc_tiling_on_sc=False, needs_layout_passes=False),
        out_type=jax.ShapeDtypeStruct((hh, sub, bt, 8 * 128), jnp.float32),
        scratch_types=[
            pltpu.VMEM((vv,), jnp.float32),
            pltpu.VMEM((2, bt, 128), jnp.int32),
            pltpu.VMEM((2, bt, 128), jnp.float32),
            pltpu.VMEM_SHARED((2, _CH, bt, 128), jnp.int32),
            pltpu.SemaphoreType.DMA,
            pltpu.SemaphoreType.DMA,
            pltpu.SemaphoreType.DMA,
            pltpu.SemaphoreType.DMA,
        ],
    )
    def grab(ids_hbm, tab_hbm, out_hbm, tab_v, idx_v, out_v, sids,
             sem_t, sem_i, sem_o, sem_s):
        c_ax = lax.axis_index("c")
        s_ax = lax.axis_index("s")
        w = s_ax * _NC + c_ax
        tr = w // 8
        r = w % 8
        lo = (_CH * s_ax) // _NS
        hi = (_CH * (s_ax + 1)) // _NS

        def idx_row(h):
            return ids_hbm.at[h // 8, :, h % 8, :]

        def stage(chunk, buf):
            def stg(i, _):
                pltpu.async_copy(idx_row(chunk * _CH + i), sids.at[buf, i],
                                 sem_s)
                return 0

            lax.fori_loop(lo, hi, stg, 0)

        def stage_sync():
            def stw(i, _):
                pltpu.make_async_copy(idx_row(0), sids.at[0, 0], sem_s).wait()
                return 0

            lax.fori_loop(lo, hi, stw, 0)
            plsc.subcore_barrier()

        def idx_fetch(buf, i, islot):
            pltpu.async_copy(sids.at[buf, i], idx_v.at[islot], sem_i)

        def idx_wait():
            pltpu.make_async_copy(sids.at[0, 0], idx_v.at[0], sem_i).wait()

        def store_wait():
            pltpu.make_async_copy(
                out_v.at[0], out_hbm.at[0, 0, :, pl.ds(0, 128)], sem_o).wait()

        pltpu.async_copy(tab_hbm.at[w], tab_v, sem_t)
        stage(0, 0)
        pltpu.make_async_copy(tab_hbm.at[w], tab_v, sem_t).wait()

        def do_h(h, buf, i, islot, first):
            idx_wait()
            if not first:
                store_wait()

            @plsc.parallel_loop(0, bt, unroll=16)
            def rowb(tc):
                for k in range(128 // _L):
                    iv = idx_v[islot, tc, pl.ds(k * _L, _L)]
                    vals = plsc.load_gather(tab_v, [iv])
                    out_v[islot, tc, pl.ds(k * _L, _L)] = vals

            @pl.when(i + 2 < _CH)
            def _():
                idx_fetch(buf, i + 2, islot)

            pltpu.async_copy(
                out_v.at[islot], out_hbm.at[h, tr, :, pl.ds(r * 128, 128)],
                sem_o)

        def do_chunk(chunk, buf, first):
            stage_sync()

            @pl.when(chunk + 1 < n_chunks)
            def _():
                stage(chunk + 1, 1 - buf)

            idx_fetch(buf, 0, 0)
            idx_fetch(buf, 1, 1)

            def pair(p, _):
                i0 = 2 * p
                h0 = chunk * _CH + i0
                do_h(h0, buf, i0, 0, first=False)
                do_h(h0 + 1, buf, i0 + 1, 1, first=False)
                return 0

            if first:
                do_h(chunk * _CH, buf, 0, 0, first=True)
                do_h(chunk * _CH + 1, buf, 1, 1, first=True)
                lax.fori_loop(1, _CH // 2, pair, 0)
            else:
                lax.fori_loop(0, _CH // 2, pair, 0)

        do_chunk(0, 0, first=True)

        def super_body(sc, _):
            do_chunk(2 * sc, 0, first=False)
            do_chunk(2 * sc + 1, 1, first=False)
            return 0

        do_chunk(1, 1, first=False)
        lax.fori_loop(1, n_chunks // 2, super_body, 0)
        store_wait()
        store_wait()

    return grab


def kernel(prompt_token_ids, table):
    b, h = prompt_token_ids.shape
    v, d = table.shape
    ids_4d = (prompt_token_ids.astype(jnp.int32).T
              .reshape(h // 8, 8, b // 128, 128).transpose(0, 2, 1, 3))
    table_t = table.T
    out = _gather_call(h, b, v, d)(ids_4d, table_t)
    out = out.reshape(h, d // 8, b // 128, 8, 128)
    return out.transpose(2, 4, 0, 1, 3).reshape(b, h, d)
